# baseline (speedup 1.0000x reference)
import jax
import jax.numpy as jnp
from jax.experimental import pallas as pl
from jax.experimental.pallas import tpu as pltpu

NUM_LEVELS = 5
KEEP_RATIO = 0.3
MIN_KEEPS = 8
EPS = 1e-5

_INTERPRET = False


def _mm(a, b):
    return jax.lax.dot_general(
        a, b, (((1,), (0,)), ((), ())),
        precision=jax.lax.Precision.HIGHEST,
        preferred_element_type=jnp.float32)


def _mmT(a, b):
    return jax.lax.dot_general(
        a, b, (((1,), (1,)), ((), ())),
        precision=jax.lax.Precision.HIGHEST,
        preferred_element_type=jnp.float32)


def _iota(shape, dim):
    return jax.lax.broadcasted_iota(jnp.int32, shape, dim)


def _pool_kernel(x_ref, o_ref):
    b = x_ref[...]
    s = 32
    while s > 1:
        b = jnp.maximum(b[:, : s // 2, :], b[:, s // 2 : s, :])
        s //= 2
    m = b[:, 0, :]
    for sh in (1, 2, 4, 8, 16):
        m = jnp.maximum(m, pltpu.roll(m, sh, axis=1))
    E = (_iota((512, 16), 0) == 32 * _iota((512, 16), 1) + 31).astype(jnp.float32)
    o_ref[...] = _mm(m, E)


def _keep_num(n):
    return min(max(MIN_KEEPS, int(n * KEEP_RATIO)), n)


def _select_kernel(cm_ref, g_ref, b_ref, w_ref, pb_ref,
                   seq_ref, crd_ref, lat_ref):
    f32 = jnp.float32
    cm4 = cm_ref[0]
    gamma = g_ref[...]
    beta = b_ref[...]
    W = w_ref[...]
    pbias = pb_ref[...]

    cm_pyr = {NUM_LEVELS - 1: cm4}
    B00 = {}
    for lvl in range(NUM_LEVELS - 2, -1, -1):
        g = 2 ** lvl
        N = g * g
        gp = 2 * g
        Np = gp * gp
        i_row = _iota((1, N), 1)
        r_col = _iota((Np, 1), 0)
        cm_acc = None
        for dy in (0, 1):
            for dx in (0, 1):
                tgt_r = (2 * (i_row // g) + dy) * gp + (2 * (i_row % g) + dx)
                Bm = (r_col == tgt_r).astype(f32)
                t = _mm(cm_pyr[lvl + 1], Bm)
                cm_acc = t if cm_acc is None else jnp.maximum(cm_acc, t)
                if dy == 0 and dx == 0:
                    B00[lvl] = Bm
        cm_pyr[lvl] = cm_acc

    n_row = {}
    for lvl in range(NUM_LEVELS):
        n_row[lvl] = jnp.sqrt(jnp.sum(cm_pyr[lvl] * cm_pyr[lvl], axis=0,
                                      keepdims=True))
    imp_row = {}
    imp_col = {}
    for lvl in range(NUM_LEVELS):
        N = 4 ** lvl
        if lvl < NUM_LEVELS - 1:
            imp_row[lvl] = jnp.abs(n_row[lvl] - _mm(n_row[lvl + 1], B00[lvl]))
        else:
            imp_row[lvl] = n_row[lvl]
        ii = _iota((N, 1), 0)
        jj = _iota((1, N), 1)
        imp_col[lvl] = jnp.sum(jnp.where(jj == ii, imp_row[lvl], 0.0),
                               axis=1, keepdims=True)

    seq_parts = []
    crd_parts = []
    lat_acc = jnp.zeros((96, 1024), f32)
    for lvl in range(NUM_LEVELS):
        g = 2 ** lvl
        N = g * g
        K = _keep_num(N)
        ic = imp_col[lvl]
        ir = imp_row[lvl]
        ii = _iota((N, 1), 0)
        jj = _iota((1, N), 1)
        Mt = ((ic > ir) | ((ic == ir) & (ii < jj))).astype(f32)
        rank_row = jnp.sum(Mt, axis=0, keepdims=True).astype(jnp.int32)

        P = cm_pyr[lvl]
        mu = jnp.mean(P, axis=0, keepdims=True)
        xc = P - mu
        var = jnp.mean(xc * xc, axis=0, keepdims=True)
        ln = xc / jnp.sqrt(var + EPS) * gamma + beta
        pf = _mm(W, ln) + pbias

        k_col = _iota((K, 1), 0)
        S = (rank_row == k_col).astype(f32)
        seq_parts.append(_mmT(S, pf))

        idxf = _mm(S, ii.astype(f32))
        gf = jnp.float32(g)
        yf = jnp.floor(idxf / gf)
        xf = idxf - yf * gf
        cx = (xf + 0.5) / gf
        cy = (yf + 0.5) / gf
        sz = jnp.full((K, 1), 1.0 / gf, f32)
        crd_parts.append(jnp.concatenate([cx, cy, sz, sz], axis=1))

        step = 32 // g
        half = 16 // g
        yj = jj // g
        xj = jj % g
        qrow = (step * yj + half) * 32 + (step * xj + half)
        keep_row = rank_row < K
        q_col = _iota((1024, 1), 0)
        Tq = ((q_col == qrow) & keep_row).astype(f32)
        lat_acc = lat_acc + _mmT(pf, Tq)

    seq_ref[0] = jnp.concatenate(seq_parts, axis=0)
    crd_ref[0] = jnp.concatenate(crd_parts, axis=0)
    lat_ref[0] = lat_acc


def _expand_kernel(lat_ref, o_ref):
    SS = lat_ref.shape[2]
    E = (_iota((32, 512), 1) == 16 * _iota((32, 512), 0)).astype(jnp.float32)
    mid = _iota((1, 16 * SS, 1), 1)
    acc = None
    for r in range(SS):
        e = _mm(lat_ref[0, :, r, 0, :], E)[:, None, :]
        part = jnp.where(mid == 16 * r, e, 0.0)
        acc = part if acc is None else acc + part
    o_ref[0] = acc


def kernel(x, ln_gamma, ln_beta, proj_w, proj_b):
    B, C, H, W = x.shape
    dtype = x.dtype
    f32 = jnp.float32

    R1 = 192
    xr = x.reshape(B * C * 16, 32, W)
    p16 = pl.pallas_call(
        _pool_kernel,
        grid=(B * C * 16 // R1,),
        in_specs=[pl.BlockSpec((R1, 32, W), lambda i: (i, 0, 0))],
        out_specs=pl.BlockSpec((R1, 16), lambda i: (i, 0)),
        out_shape=jax.ShapeDtypeStruct((B * C * 16, 16), f32),
        compiler_params=pltpu.CompilerParams(
            dimension_semantics=("parallel",)),
        interpret=_INTERPRET,
    )(xr)

    cm = p16.reshape(B, C, 256)
    totK = sum(_keep_num(4 ** l) for l in range(NUM_LEVELS))

    seq, crd, lat = pl.pallas_call(
        _select_kernel,
        grid=(B,),
        in_specs=[
            pl.BlockSpec((1, C, 256), lambda b: (b, 0, 0)),
            pl.BlockSpec((C, 1), lambda b: (0, 0)),
            pl.BlockSpec((C, 1), lambda b: (0, 0)),
            pl.BlockSpec((C, C), lambda b: (0, 0)),
            pl.BlockSpec((C, 1), lambda b: (0, 0)),
        ],
        out_specs=[
            pl.BlockSpec((1, totK, C), lambda b: (b, 0, 0)),
            pl.BlockSpec((1, totK, 4), lambda b: (b, 0, 0)),
            pl.BlockSpec((1, C, 1024), lambda b: (b, 0, 0)),
        ],
        out_shape=[
            jax.ShapeDtypeStruct((B, totK, C), f32),
            jax.ShapeDtypeStruct((B, totK, 4), f32),
            jax.ShapeDtypeStruct((B, C, 1024), f32),
        ],
        compiler_params=pltpu.CompilerParams(
            dimension_semantics=("parallel",)),
        interpret=_INTERPRET,
    )(cm, ln_gamma.reshape(C, 1), ln_beta.reshape(C, 1),
      proj_w, proj_b.reshape(C, 1))

    SS = 8
    lat5 = lat.reshape(B, C, 32, 1, 32)
    out_sparse = pl.pallas_call(
        _expand_kernel,
        grid=(B, 32 // SS),
        in_specs=[pl.BlockSpec((1, C, SS, 1, 32), lambda b, s: (b, 0, s, 0, 0))],
        out_specs=pl.BlockSpec((1, C, 16 * SS, W), lambda b, s: (b, 0, s, 0)),
        out_shape=jax.ShapeDtypeStruct((B, C, H, W), f32),
        compiler_params=pltpu.CompilerParams(
            dimension_semantics=("parallel", "parallel")),
        interpret=_INTERPRET,
    )(lat5)

    sparsity = jnp.asarray(totK / (H * W), dtype)
    return (out_sparse.astype(dtype), seq.astype(dtype),
            crd.astype(dtype), sparsity)


# speedup vs baseline: 1.0695x; 1.0695x over previous
import jax
import jax.numpy as jnp
from jax.experimental import pallas as pl
from jax.experimental.pallas import tpu as pltpu

NUM_LEVELS = 5
KEEP_RATIO = 0.3
MIN_KEEPS = 8
EPS = 1e-5

_INTERPRET = False


def _mm(a, b):
    return jax.lax.dot_general(
        a, b, (((1,), (0,)), ((), ())),
        precision=jax.lax.Precision.HIGHEST,
        preferred_element_type=jnp.float32)


def _mmT(a, b):
    return jax.lax.dot_general(
        a, b, (((1,), (1,)), ((), ())),
        precision=jax.lax.Precision.HIGHEST,
        preferred_element_type=jnp.float32)


def _iota(shape, dim):
    return jax.lax.broadcasted_iota(jnp.int32, shape, dim)


def _pool_kernel(x_ref, o_ref):
    b = x_ref[...]
    s = 32
    while s > 1:
        b = jnp.maximum(b[:, : s // 2, :], b[:, s // 2 : s, :])
        s //= 2
    m = b[:, 0, :]
    for sh in (1, 2, 4, 8, 16):
        m = jnp.maximum(m, pltpu.roll(m, sh, axis=1))
    E = (_iota((512, 16), 0) == 32 * _iota((512, 16), 1) + 31).astype(jnp.float32)
    o_ref[...] = _mm(m, E)


def _keep_num(n):
    return min(max(MIN_KEEPS, int(n * KEEP_RATIO)), n)


def _select_body(cm_ref, g_ref, b_ref, w_ref, pb_ref, seq_ref, crd_ref,
                 lat_scr):
    f32 = jnp.float32
    cm4 = cm_ref[0]
    gamma = g_ref[...]
    beta = b_ref[...]
    W = w_ref[...]
    pbias = pb_ref[...]

    cm_pyr = {NUM_LEVELS - 1: cm4}
    B00 = {}
    for lvl in range(NUM_LEVELS - 2, -1, -1):
        g = 2 ** lvl
        N = g * g
        gp = 2 * g
        Np = gp * gp
        i_row = _iota((1, N), 1)
        r_col = _iota((Np, 1), 0)
        cm_acc = None
        for dy in (0, 1):
            for dx in (0, 1):
                tgt_r = (2 * (i_row // g) + dy) * gp + (2 * (i_row % g) + dx)
                Bm = (r_col == tgt_r).astype(f32)
                t = _mm(cm_pyr[lvl + 1], Bm)
                cm_acc = t if cm_acc is None else jnp.maximum(cm_acc, t)
                if dy == 0 and dx == 0:
                    B00[lvl] = Bm
        cm_pyr[lvl] = cm_acc

    n_row = {}
    for lvl in range(NUM_LEVELS):
        n_row[lvl] = jnp.sqrt(jnp.sum(cm_pyr[lvl] * cm_pyr[lvl], axis=0,
                                      keepdims=True))
    imp_row = {}
    imp_col = {}
    for lvl in range(NUM_LEVELS):
        N = 4 ** lvl
        if lvl < NUM_LEVELS - 1:
            imp_row[lvl] = jnp.abs(n_row[lvl] - _mm(n_row[lvl + 1], B00[lvl]))
        else:
            imp_row[lvl] = n_row[lvl]
        ii = _iota((N, 1), 0)
        jj = _iota((1, N), 1)
        imp_col[lvl] = jnp.sum(jnp.where(jj == ii, imp_row[lvl], 0.0),
                               axis=1, keepdims=True)

    seq_parts = []
    crd_parts = []
    lat_acc = jnp.zeros((96, 1024), f32)
    for lvl in range(NUM_LEVELS):
        g = 2 ** lvl
        N = g * g
        K = _keep_num(N)
        ic = imp_col[lvl]
        ir = imp_row[lvl]
        ii = _iota((N, 1), 0)
        jj = _iota((1, N), 1)
        Mt = ((ic > ir) | ((ic == ir) & (ii < jj))).astype(f32)
        rank_row = jnp.sum(Mt, axis=0, keepdims=True).astype(jnp.int32)

        P = cm_pyr[lvl]
        mu = jnp.mean(P, axis=0, keepdims=True)
        xc = P - mu
        var = jnp.mean(xc * xc, axis=0, keepdims=True)
        ln = xc / jnp.sqrt(var + EPS) * gamma + beta
        pf = _mm(W, ln) + pbias

        k_col = _iota((K, 1), 0)
        S = (rank_row == k_col).astype(f32)
        seq_parts.append(_mmT(S, pf))

        idxf = _mm(S, ii.astype(f32))
        gf = jnp.float32(g)
        yf = jnp.floor(idxf / gf)
        xf = idxf - yf * gf
        cx = (xf + 0.5) / gf
        cy = (yf + 0.5) / gf
        sz = jnp.full((K, 1), 1.0 / gf, f32)
        crd_parts.append(jnp.concatenate([cx, cy, sz, sz], axis=1))

        step = 32 // g
        half = 16 // g
        yj = jj // g
        xj = jj % g
        qrow = (step * yj + half) * 32 + (step * xj + half)
        keep_row = rank_row < K
        q_col = _iota((1024, 1), 0)
        Tq = ((q_col == qrow) & keep_row).astype(f32)
        lat_acc = lat_acc + _mmT(pf, Tq)

    seq_ref[0] = jnp.concatenate(seq_parts, axis=0)
    crd_ref[0] = jnp.concatenate(crd_parts, axis=0)
    for ky in range(32):
        lat_scr[ky] = lat_acc[:, 32 * ky : 32 * ky + 32]


def _select_expand_kernel(cm_ref, g_ref, b_ref, w_ref, pb_ref,
                          seq_ref, crd_ref, o_ref, lat_scr):
    s = pl.program_id(1)

    @pl.when(s == 0)
    def _():
        _select_body(cm_ref, g_ref, b_ref, w_ref, pb_ref, seq_ref, crd_ref,
                     lat_scr)

    SS = o_ref.shape[2] // 16
    E = (_iota((32, 512), 1) == 16 * _iota((32, 512), 0)).astype(jnp.float32)
    mid = _iota((1, 16 * SS, 1), 1)
    acc = None
    for r in range(SS):
        e = _mm(lat_scr[SS * s + r], E)[:, None, :]
        part = jnp.where(mid == 16 * r, e, 0.0)
        acc = part if acc is None else acc + part
    o_ref[0] = acc


def kernel(x, ln_gamma, ln_beta, proj_w, proj_b):
    B, C, H, W = x.shape
    dtype = x.dtype
    f32 = jnp.float32

    R1 = 128
    xr = x.reshape(B * C * 16, 32, W)
    p16 = pl.pallas_call(
        _pool_kernel,
        grid=(B * C * 16 // R1,),
        in_specs=[pl.BlockSpec((R1, 32, W), lambda i: (i, 0, 0))],
        out_specs=pl.BlockSpec((R1, 16), lambda i: (i, 0)),
        out_shape=jax.ShapeDtypeStruct((B * C * 16, 16), f32),
        compiler_params=pltpu.CompilerParams(
            dimension_semantics=("parallel",)),
        interpret=_INTERPRET,
    )(xr)

    cm = p16.reshape(B, C, 256)
    totK = sum(_keep_num(4 ** l) for l in range(NUM_LEVELS))
    SS = 4

    seq, crd, out_sparse = pl.pallas_call(
        _select_expand_kernel,
        grid=(B, 32 // SS),
        in_specs=[
            pl.BlockSpec((1, C, 256), lambda b, s: (b, 0, 0)),
            pl.BlockSpec((C, 1), lambda b, s: (0, 0)),
            pl.BlockSpec((C, 1), lambda b, s: (0, 0)),
            pl.BlockSpec((C, C), lambda b, s: (0, 0)),
            pl.BlockSpec((C, 1), lambda b, s: (0, 0)),
        ],
        out_specs=[
            pl.BlockSpec((1, totK, C), lambda b, s: (b, 0, 0)),
            pl.BlockSpec((1, totK, 4), lambda b, s: (b, 0, 0)),
            pl.BlockSpec((1, C, 16 * SS, W), lambda b, s: (b, 0, s, 0)),
        ],
        out_shape=[
            jax.ShapeDtypeStruct((B, totK, C), f32),
            jax.ShapeDtypeStruct((B, totK, 4), f32),
            jax.ShapeDtypeStruct((B, C, H, W), f32),
        ],
        scratch_shapes=[pltpu.VMEM((32, C, 32), f32)],
        compiler_params=pltpu.CompilerParams(
            dimension_semantics=("arbitrary", "arbitrary")),
        interpret=_INTERPRET,
    )(cm, ln_gamma.reshape(C, 1), ln_beta.reshape(C, 1),
      proj_w, proj_b.reshape(C, 1))

    sparsity = jnp.asarray(totK / (H * W), dtype)
    return (out_sparse.astype(dtype), seq.astype(dtype),
            crd.astype(dtype), sparsity)
